# baseline (speedup 1.0000x reference)
import functools

import jax
import jax.numpy as jnp
from jax import lax
from jax.experimental import pallas as pl
from jax.experimental.pallas import tpu as pltpu
from jax.experimental.pallas import tpu_sc as plsc

N_NODES = 10000
N_EDGES = 320000
D = 128

NC = 2
NS = 16
NW = NC * NS
CH = 128
NCH = 80
NCH2 = 8
E_PAD = NW * NCH * CH
N_PAD = 10240
NZCH = N_PAD // (NS * CH)
ZROWS = N_PAD // NS
ND = N_PAD // 8
NDCH = ND // CH


def _sc_body(x_hbm, src_hbm, dst_hbm, dhi_hbm, dlo_hbm, oh_hbm, zrows_hbm,
             zidx_hbm, zidx2_hbm, aggp_hbm, degp_hbm,
             src_v, dst_v, dhi_v, dlo_v, rows_v, zidx_v, zidx2_v, sem,
             sacc, sdeg):
    cid = lax.axis_index("c")
    sid = lax.axis_index("s")
    wid = cid * NS + sid

    pltpu.sync_copy(zidx_hbm.at[sid], zidx_v)
    pltpu.sync_copy(zidx2_hbm.at[sid], zidx2_v)
    pltpu.sync_copy(zrows_hbm, rows_v)
    for t in range(NZCH):
        pltpu.sync_copy(rows_v, sacc.at[zidx_v.at[t]])

    @pl.when(sid < NDCH)
    def _():
        pltpu.sync_copy(rows_v, sdeg.at[zidx2_v.at[0]])

    plsc.subcore_barrier()

    def step(j, carry):
        pltpu.async_copy(x_hbm.at[src_v.at[j]], rows_v, sem).wait()
        pltpu.sync_copy(rows_v, sacc.at[dst_v.at[j]], add=True)
        pltpu.async_copy(oh_hbm.at[dlo_v.at[j]], rows_v, sem).wait()
        pltpu.sync_copy(rows_v, sdeg.at[dhi_v.at[j]], add=True)
        return carry

    for p in range(NCH // NCH2):
        pltpu.sync_copy(src_hbm.at[wid, pl.ds(p * NCH2, NCH2)], src_v)
        pltpu.sync_copy(dst_hbm.at[wid, pl.ds(p * NCH2, NCH2)], dst_v)
        pltpu.sync_copy(dhi_hbm.at[wid, pl.ds(p * NCH2, NCH2)], dhi_v)
        pltpu.sync_copy(dlo_hbm.at[wid, pl.ds(p * NCH2, NCH2)], dlo_v)
        lax.fori_loop(0, NCH2, step, 0)

    plsc.subcore_barrier()

    for t in range(NZCH):
        o = pl.multiple_of(sid * ZROWS + t * CH, 8)
        pltpu.async_copy(sacc.at[zidx_v.at[t]], rows_v, sem).wait()
        pltpu.sync_copy(rows_v, aggp_hbm.at[cid, pl.ds(o, CH)])

    @pl.when(sid < NDCH)
    def _():
        o2 = pl.multiple_of(sid * CH, 8)
        pltpu.async_copy(sdeg.at[zidx2_v.at[0]], rows_v, sem).wait()
        pltpu.sync_copy(rows_v, degp_hbm.at[cid, pl.ds(o2, CH)])


@functools.cache
def _sc_aggregate():
  return pl.kernel(
    _sc_body,
    out_type=(
        jax.ShapeDtypeStruct((NC, N_PAD, D), jnp.float32),
        jax.ShapeDtypeStruct((NC, ND, D), jnp.float32),
    ),
    mesh=plsc.VectorSubcoreMesh(core_axis_name="c", subcore_axis_name="s"),
    scratch_types=[
        pltpu.VMEM((NCH2, CH), jnp.int32),
        pltpu.VMEM((NCH2, CH), jnp.int32),
        pltpu.VMEM((NCH2, CH), jnp.int32),
        pltpu.VMEM((NCH2, CH), jnp.int32),
        pltpu.VMEM((CH, D), jnp.float32),
        pltpu.VMEM((NZCH, CH), jnp.int32),
        pltpu.VMEM((1, CH), jnp.int32),
        pltpu.SemaphoreType.DMA,
        pltpu.VMEM_SHARED((N_PAD, D), jnp.float32),
        pltpu.VMEM_SHARED((ND, D), jnp.float32),
    ],
  )


BLK = 1000


def _tc_body(aggp_ref, degp_ref, w_ref, b_ref, gamma_ref, beta_ref, out_ref):
    agg = aggp_ref[0] + aggp_ref[1]
    deg = (degp_ref[0, 0] + degp_ref[0, 1]).reshape(BLK, 1)
    mean = agg / jnp.maximum(deg, 1.0)
    h = jnp.dot(mean, w_ref[...], preferred_element_type=jnp.float32)
    h = h + b_ref[...]
    mu = jnp.mean(h, axis=1, keepdims=True)
    var = jnp.mean((h - mu) * (h - mu), axis=1, keepdims=True)
    hn = (h - mu) * lax.rsqrt(var + 1e-5) * gamma_ref[...] + beta_ref[...]
    out_ref[...] = jnp.maximum(hn, 0.0)


_tc_finish = pl.pallas_call(
    _tc_body,
    grid=(N_NODES // BLK,),
    in_specs=[
        pl.BlockSpec((NC, BLK, D), lambda i: (0, i, 0)),
        pl.BlockSpec((1, NC, BLK), lambda i: (i, 0, 0)),
        pl.BlockSpec((D, D), lambda i: (0, 0)),
        pl.BlockSpec((1, D), lambda i: (0, 0)),
        pl.BlockSpec((1, D), lambda i: (0, 0)),
        pl.BlockSpec((1, D), lambda i: (0, 0)),
    ],
    out_specs=pl.BlockSpec((BLK, D), lambda i: (i, 0)),
    out_shape=jax.ShapeDtypeStruct((N_NODES, D), jnp.float32),
)


@jax.jit
def kernel(x, edge_index, W, b, gamma, beta):
    ei = edge_index.astype(jnp.int32)
    pad = E_PAD - N_EDGES
    src = jnp.concatenate([ei[0], jnp.zeros((pad,), jnp.int32)])
    dst = jnp.concatenate([ei[1], jnp.full((pad,), N_NODES, jnp.int32)])
    src = src.reshape(NW, NCH, CH)
    dst = dst.reshape(NW, NCH, CH)
    dhi = dst >> 3
    dlo = dst & 7
    oh = (jnp.arange(D, dtype=jnp.int32) // 16 ==
          jnp.arange(8, dtype=jnp.int32)[:, None]).astype(jnp.float32)
    zrows = jnp.zeros((CH, D), jnp.float32)
    zidx = jnp.arange(N_PAD, dtype=jnp.int32).reshape(NS, NZCH, CH)
    zidx2 = jnp.arange(NS * CH, dtype=jnp.int32).reshape(NS, 1, CH) % ND

    aggp, degp = _sc_aggregate()(x, src, dst, dhi, dlo, oh, zrows,
                                 zidx, zidx2)
    deg = degp.reshape(NC, ND, 8, 16)[:, :, :, 0].reshape(NC, N_PAD)
    deg = deg[:, :N_NODES].reshape(NC, N_NODES // BLK, BLK).transpose(1, 0, 2)
    return _tc_finish(aggp, deg, W, b.reshape(1, D),
                      gamma.reshape(1, D), beta.reshape(1, D))


# speedup vs baseline: 2.1220x; 2.1220x over previous
import functools

import jax
import jax.numpy as jnp
from jax import lax
from jax.experimental import pallas as pl
from jax.experimental.pallas import tpu as pltpu
from jax.experimental.pallas import tpu_sc as plsc

N_NODES = 10000
N_EDGES = 320000
D = 128

NC = 2
NS = 16
NW = NC * NS
CH = 128
NCH = 80
NCH2 = 8
E_PAD = NW * NCH * CH
N_PAD = 10240
NZCH = N_PAD // (NS * CH)
ZROWS = N_PAD // NS
ND = N_PAD // 8
NDCH = ND // CH
NOH = 1024


def _sc_body(x_hbm, src_hbm, dst_hbm, dhi_hbm, dlo_hbm, oh_hbm, zrows_hbm,
             zidx_hbm, zidx2_hbm, aggp_hbm, degp_hbm,
             src_v, dst_v, dhi_v, dlo_v, rows_v, zidx_v, zidx2_v, sem,
             sacc, sdeg):
    cid = lax.axis_index("c")
    sid = lax.axis_index("s")
    wid = cid * NS + sid

    pltpu.sync_copy(zidx_hbm.at[sid], zidx_v)
    pltpu.sync_copy(zidx2_hbm.at[sid], zidx2_v)
    pltpu.sync_copy(zrows_hbm, rows_v)
    for t in range(NZCH):
        pltpu.sync_copy(rows_v, sacc.at[zidx_v.at[t]])

    @pl.when(sid < NDCH)
    def _():
        pltpu.sync_copy(rows_v, sdeg.at[zidx2_v.at[0]])

    plsc.subcore_barrier()

    def step(j, carry):
        pltpu.async_copy(x_hbm.at[src_v.at[j]], rows_v, sem).wait()
        pltpu.sync_copy(rows_v, sacc.at[dst_v.at[j]], add=True)
        pltpu.async_copy(oh_hbm.at[dlo_v.at[j]], rows_v, sem).wait()
        pltpu.sync_copy(rows_v, sdeg.at[dhi_v.at[j]], add=True)
        return carry

    for p in range(NCH // NCH2):
        pltpu.sync_copy(src_hbm.at[wid, pl.ds(p * NCH2, NCH2)], src_v)
        pltpu.sync_copy(dst_hbm.at[wid, pl.ds(p * NCH2, NCH2)], dst_v)
        pltpu.sync_copy(dhi_hbm.at[wid, pl.ds(p * NCH2, NCH2)], dhi_v)
        pltpu.sync_copy(dlo_hbm.at[wid, pl.ds(p * NCH2, NCH2)], dlo_v)
        lax.fori_loop(0, NCH2, step, 0)

    plsc.subcore_barrier()

    for t in range(NZCH):
        o = pl.multiple_of(sid * ZROWS + t * CH, 8)
        pltpu.async_copy(sacc.at[zidx_v.at[t]], rows_v, sem).wait()
        pltpu.sync_copy(rows_v, aggp_hbm.at[cid, pl.ds(o, CH)])

    @pl.when(sid < NDCH)
    def _():
        o2 = pl.multiple_of(sid * CH, 8)
        pltpu.async_copy(sdeg.at[zidx2_v.at[0]], rows_v, sem).wait()
        pltpu.sync_copy(rows_v, degp_hbm.at[cid, pl.ds(o2, CH)])


@functools.cache
def _sc_aggregate():
  return pl.kernel(
    _sc_body,
    out_type=(
        jax.ShapeDtypeStruct((NC, N_PAD, D), jnp.float32),
        jax.ShapeDtypeStruct((NC, ND, D), jnp.float32),
    ),
    mesh=plsc.VectorSubcoreMesh(core_axis_name="c", subcore_axis_name="s"),
    scratch_types=[
        pltpu.VMEM((NCH2, CH), jnp.int32),
        pltpu.VMEM((NCH2, CH), jnp.int32),
        pltpu.VMEM((NCH2, CH), jnp.int32),
        pltpu.VMEM((NCH2, CH), jnp.int32),
        pltpu.VMEM((CH, D), jnp.float32),
        pltpu.VMEM((NZCH, CH), jnp.int32),
        pltpu.VMEM((1, CH), jnp.int32),
        pltpu.SemaphoreType.DMA,
        pltpu.VMEM_SHARED((N_PAD, D), jnp.float32),
        pltpu.VMEM_SHARED((ND, D), jnp.float32),
    ],
  )


BLK = 1000


def _tc_body(aggp_ref, degp_ref, w_ref, b_ref, gamma_ref, beta_ref, out_ref):
    agg = aggp_ref[0] + aggp_ref[1]
    deg = (degp_ref[0, 0] + degp_ref[0, 1]).reshape(BLK, 1)
    mean = agg / jnp.maximum(deg, 1.0)
    h = jnp.dot(mean, w_ref[...], preferred_element_type=jnp.float32)
    h = h + b_ref[...]
    mu = jnp.mean(h, axis=1, keepdims=True)
    var = jnp.mean((h - mu) * (h - mu), axis=1, keepdims=True)
    hn = (h - mu) * lax.rsqrt(var + 1e-5) * gamma_ref[...] + beta_ref[...]
    out_ref[...] = jnp.maximum(hn, 0.0)


_tc_finish = pl.pallas_call(
    _tc_body,
    grid=(N_NODES // BLK,),
    in_specs=[
        pl.BlockSpec((NC, BLK, D), lambda i: (0, i, 0)),
        pl.BlockSpec((1, NC, BLK), lambda i: (i, 0, 0)),
        pl.BlockSpec((D, D), lambda i: (0, 0)),
        pl.BlockSpec((1, D), lambda i: (0, 0)),
        pl.BlockSpec((1, D), lambda i: (0, 0)),
        pl.BlockSpec((1, D), lambda i: (0, 0)),
    ],
    out_specs=pl.BlockSpec((BLK, D), lambda i: (i, 0)),
    out_shape=jax.ShapeDtypeStruct((N_NODES, D), jnp.float32),
)


@jax.jit
def kernel(x, edge_index, W, b, gamma, beta):
    ei = edge_index.astype(jnp.int32)
    pad = E_PAD - N_EDGES
    src = jnp.concatenate([ei[0], jnp.zeros((pad,), jnp.int32)])
    dst = jnp.concatenate([ei[1], jnp.full((pad,), N_NODES, jnp.int32)])
    src = src.reshape(NW, NCH, CH)
    dst = dst.reshape(NW, NCH, CH)
    dhi = dst >> 3
    dlo = dst & (NOH - 1)
    oh = (jnp.arange(D, dtype=jnp.int32) // 16 ==
          (jnp.arange(NOH, dtype=jnp.int32) & 7)[:, None]).astype(jnp.float32)
    zrows = jnp.zeros((CH, D), jnp.float32)
    zidx = jnp.arange(N_PAD, dtype=jnp.int32).reshape(NS, NZCH, CH)
    zidx2 = jnp.arange(NS * CH, dtype=jnp.int32).reshape(NS, 1, CH) % ND

    aggp, degp = _sc_aggregate()(x, src, dst, dhi, dlo, oh, zrows,
                                 zidx, zidx2)
    deg = degp.reshape(NC, ND, 8, 16)[:, :, :, 0].reshape(NC, N_PAD)
    deg = deg[:, :N_NODES].reshape(NC, N_NODES // BLK, BLK).transpose(1, 0, 2)
    return _tc_finish(aggp, deg, W, b.reshape(1, D),
                      gamma.reshape(1, D), beta.reshape(1, D))
